# Initial kernel scaffold; baseline (speedup 1.0000x reference)
import jax
import jax.numpy as jnp
from jax.experimental import pallas as pl


def kernel(x, edge_index, edge_attr, W, att_src, att_dst, W_edge, att_edge, bias_conv, W_lin, b_lin):
    raise NotImplementedError("write your pallas kernel here")


# speedup vs baseline: 21.1262x; 21.1262x over previous
import functools

import jax
import jax.numpy as jnp
from jax import lax
from jax.experimental import pallas as pl
from jax.experimental.pallas import tpu as pltpu
from jax.experimental.pallas import tpu_sc as plsc

NC = 2
NS = 16
NW = NC * NS


def _tc1_body(x_ref, w_ref, a_ref, s_ref, aux_ref):
    i = pl.program_id(0)
    v = jnp.dot(w_ref[...], a_ref[...], preferred_element_type=jnp.float32,
                precision=lax.Precision.HIGHEST)
    s = jnp.dot(x_ref[...], v, preferred_element_type=jnp.float32,
                precision=lax.Precision.HIGHEST)
    s_ref[...] = s
    m = jnp.max(s, axis=0).reshape(8, 1)
    mb = jnp.broadcast_to(m, (8, 128))

    @pl.when(i == 0)
    def _():
        aux_ref[...] = mb

    @pl.when(i > 0)
    def _():
        aux_ref[...] = jnp.maximum(aux_ref[...], mb)


def _tc2_body(ea_ref, we_ref, ae_ref, out_ref, aux_ref):
    i = pl.program_id(0)
    ve = jnp.dot(we_ref[...], ae_ref[...], preferred_element_type=jnp.float32,
                precision=lax.Precision.HIGHEST)
    a = jnp.dot(ea_ref[...], ve, preferred_element_type=jnp.float32,
                precision=lax.Precision.HIGHEST)
    out_ref[...] = a
    mb = jnp.broadcast_to(jnp.max(a), (8, 128))

    @pl.when(i == 0)
    def _():
        aux_ref[...] = mb

    @pl.when(i > 0)
    def _():
        aux_ref[...] = jnp.maximum(aux_ref[...], mb)


def _tc3_body(num_ref, den_ref, bc_ref, wl_ref, bl_ref, out_ref):
    ns = jnp.sum(num_ref[...], axis=0, keepdims=True)
    ds_ = jnp.sum(den_ref[...], axis=0, keepdims=True)
    c = jnp.sum(bc_ref[...] * wl_ref[...]) + bl_ref[0, 0]
    agg = jnp.where(ds_ > 0.0, ns / ds_, 0.0)
    out_ref[...] = jnp.maximum(agg + c, 0.0)


def _make_sc_kernel(n, per):
    mesh = plsc.VectorSubcoreMesh(core_axis_name="c", subcore_axis_name="s")

    @functools.partial(
        pl.kernel,
        mesh=mesh,
        compiler_params=pltpu.CompilerParams(needs_layout_passes=False),
        out_type=[
            jax.ShapeDtypeStruct((NW, n), jnp.float32),
            jax.ShapeDtypeStruct((NW, n), jnp.float32),
        ],
        scratch_types=[
            pltpu.VMEM((n,), jnp.float32),
            pltpu.VMEM((n,), jnp.float32),
            pltpu.VMEM((n,), jnp.float32),
            pltpu.VMEM((per,), jnp.int32),
            pltpu.VMEM((per,), jnp.int32),
            pltpu.VMEM((per,), jnp.float32),
            pltpu.VMEM((16,), jnp.float32),
            pltpu.VMEM((n,), jnp.float32),
            pltpu.VMEM((n,), jnp.float32),
        ],
    )
    def sc_kernel(asrc_hbm, adst_hbm, z_hbm, src_hbm, dst_hbm, ae_hbm,
                  cvec_hbm, num_out, den_out,
                  asrc_v, adst_v, z_v, src_v, dst_v, ae_v, cv_v, num_v, den_v):
        cid = lax.axis_index("c")
        sid = lax.axis_index("s")
        wid = sid * NC + cid
        base = wid * per
        pltpu.sync_copy(asrc_hbm, asrc_v)
        pltpu.sync_copy(adst_hbm, adst_v)
        pltpu.sync_copy(z_hbm, z_v)
        pltpu.sync_copy(src_hbm.at[pl.ds(base, per)], src_v)
        pltpu.sync_copy(dst_hbm.at[pl.ds(base, per)], dst_v)
        pltpu.sync_copy(ae_hbm.at[pl.ds(base, per)], ae_v)
        pltpu.sync_copy(cvec_hbm, cv_v)
        c16 = cv_v[...]

        def zbody(i, carry):
            num_v[pl.ds(i * 16, 16)] = jnp.zeros((16,), jnp.float32)
            den_v[pl.ds(i * 16, 16)] = jnp.zeros((16,), jnp.float32)
            return carry

        lax.fori_loop(0, n // 16, zbody, 0)

        def ebody(i, carry):
            off = i * 16
            s16 = src_v[pl.ds(off, 16)]
            d16 = dst_v[pl.ds(off, 16)]
            ae16 = ae_v[pl.ds(off, 16)]
            asr = plsc.load_gather(asrc_v, [s16])
            ads = plsc.load_gather(adst_v, [d16])
            z16 = plsc.load_gather(z_v, [s16])
            al = asr + ads + ae16
            al = jnp.where(al >= 0.0, al, al * 0.2)
            ex = jnp.exp(al - c16)
            plsc.addupdate_scatter(den_v, [d16], ex)
            plsc.addupdate_scatter(num_v, [d16], ex * z16)
            return carry

        lax.fori_loop(0, per // 16, ebody, 0)
        pltpu.sync_copy(num_v, num_out.at[wid])
        pltpu.sync_copy(den_v, den_out.at[wid])

    return sc_kernel


def kernel(x, edge_index, edge_attr, W, att_src, att_dst, W_edge, att_edge,
           bias_conv, W_lin, b_lin):
    n, d_in = x.shape
    e = edge_index.shape[1]
    d_edge = edge_attr.shape[1]
    per = (-(-e // NW) + 15) // 16 * 16
    e_pad = per * NW

    a_mat = jnp.concatenate(
        [att_src[:, None], att_dst[:, None], W_lin,
         jnp.zeros((d_in, 5), jnp.float32)], axis=1)
    blk_n = 1000
    s_out, aux1 = pl.pallas_call(
        _tc1_body,
        grid=(n // blk_n,),
        in_specs=[
            pl.BlockSpec((blk_n, d_in), lambda i: (i, 0)),
            pl.BlockSpec((d_in, d_in), lambda i: (0, 0)),
            pl.BlockSpec((d_in, 8), lambda i: (0, 0)),
        ],
        out_specs=[
            pl.BlockSpec((blk_n, 8), lambda i: (i, 0)),
            pl.BlockSpec((8, 128), lambda i: (0, 0)),
        ],
        out_shape=[
            jax.ShapeDtypeStruct((n, 8), jnp.float32),
            jax.ShapeDtypeStruct((8, 128), jnp.float32),
        ],
    )(x, W, a_mat)

    blk_e = 8000
    ae_out, aux2 = pl.pallas_call(
        _tc2_body,
        grid=(e // blk_e,),
        in_specs=[
            pl.BlockSpec((blk_e, d_edge), lambda i: (i, 0)),
            pl.BlockSpec((d_edge, d_in), lambda i: (0, 0)),
            pl.BlockSpec((d_in, 1), lambda i: (0, 0)),
        ],
        out_specs=[
            pl.BlockSpec((blk_e, 1), lambda i: (i, 0)),
            pl.BlockSpec((8, 128), lambda i: (0, 0)),
        ],
        out_shape=[
            jax.ShapeDtypeStruct((e, 1), jnp.float32),
            jax.ShapeDtypeStruct((8, 128), jnp.float32),
        ],
    )(edge_attr, W_edge, att_edge[:, None])

    a_src = s_out[:, 0]
    a_dst = s_out[:, 1]
    z = s_out[:, 2]
    c_shift = aux1[0, 0] + aux1[1, 0] + aux2[0, 0]
    cvec = jnp.broadcast_to(c_shift, (16,)).astype(jnp.float32)
    pad = e_pad - e
    srcp = jnp.concatenate([edge_index[0], jnp.zeros((pad,), jnp.int32)])
    dstp = jnp.concatenate([edge_index[1], jnp.zeros((pad,), jnp.int32)])
    aep = jnp.concatenate(
        [ae_out.reshape(e), jnp.full((pad,), -1e30, jnp.float32)])

    num_p, den_p = _make_sc_kernel(n, per)(
        a_src, a_dst, z, srcp, dstp, aep, cvec)

    out = pl.pallas_call(
        _tc3_body,
        out_shape=jax.ShapeDtypeStruct((1, n), jnp.float32),
    )(num_p, den_p, bias_conv.reshape(3, 128), W_lin.reshape(3, 128),
      b_lin.reshape(1, 1))
    return out.reshape(n, 1)


# speedup vs baseline: 33.2257x; 1.5727x over previous
import functools

import jax
import jax.numpy as jnp
from jax import lax
from jax.experimental import pallas as pl
from jax.experimental.pallas import tpu as pltpu
from jax.experimental.pallas import tpu_sc as plsc

NC = 2
NS = 16
NW = NC * NS


def _tc1_body(x_ref, w_ref, a_ref, s_ref, aux_ref):
    i = pl.program_id(0)
    v = jnp.dot(w_ref[...], a_ref[...], preferred_element_type=jnp.float32)
    s = jnp.dot(x_ref[...], v, preferred_element_type=jnp.float32)
    s_ref[...] = s
    m = jnp.max(s, axis=0).reshape(8, 1)
    mb = jnp.broadcast_to(m, (8, 128))

    @pl.when(i == 0)
    def _():
        aux_ref[...] = mb

    @pl.when(i > 0)
    def _():
        aux_ref[...] = jnp.maximum(aux_ref[...], mb)


def _tc2_body(ea_ref, we_ref, ae_ref, out_ref, aux_ref):
    i = pl.program_id(0)
    ve = jnp.dot(we_ref[...], ae_ref[...], preferred_element_type=jnp.float32)
    vt = jnp.broadcast_to(ve.reshape(1, 16, 1), (8, 16, 1)).reshape(128, 1)
    row = lax.broadcasted_iota(jnp.int32, (128, 8), 0)
    col = lax.broadcasted_iota(jnp.int32, (128, 8), 1)
    vd = jnp.where(row // 16 == col, vt, 0.0)
    a = jnp.dot(ea_ref[...], vd, preferred_element_type=jnp.float32)
    out_ref[...] = a
    mb = jnp.broadcast_to(jnp.max(a), (8, 128))

    @pl.when(i == 0)
    def _():
        aux_ref[...] = mb

    @pl.when(i > 0)
    def _():
        aux_ref[...] = jnp.maximum(aux_ref[...], mb)


def _tc3_body(num_ref, den_ref, bc_ref, wl_ref, bl_ref, out_ref):
    ns = jnp.sum(num_ref[...], axis=0, keepdims=True)
    ds_ = jnp.sum(den_ref[...], axis=0, keepdims=True)
    c = jnp.sum(bc_ref[...] * wl_ref[...]) + bl_ref[0, 0]
    agg = jnp.where(ds_ > 0.0, ns / ds_, 0.0)
    out_ref[...] = jnp.maximum(agg + c, 0.0)


def _make_sc_kernel(n, e):
    per = -(-e // NW)
    nfull = per // 16
    tail = per - nfull * 16
    buf = (per + 15) // 16 * 16
    mesh = plsc.VectorSubcoreMesh(core_axis_name="c", subcore_axis_name="s")

    @functools.partial(
        pl.kernel,
        mesh=mesh,
        compiler_params=pltpu.CompilerParams(needs_layout_passes=False),
        out_type=[
            jax.ShapeDtypeStruct((NW, n), jnp.float32),
            jax.ShapeDtypeStruct((NW, n), jnp.float32),
        ],
        scratch_types=[
            pltpu.VMEM((8 * n,), jnp.float32),
            pltpu.VMEM((buf,), jnp.int32),
            pltpu.VMEM((buf,), jnp.int32),
            pltpu.VMEM((buf,), jnp.float32),
            pltpu.VMEM((16,), jnp.float32),
            pltpu.VMEM((n,), jnp.float32),
            pltpu.VMEM((n,), jnp.float32),
        ],
    )
    def sc_kernel(s_hbm, ei_hbm, ae_hbm, cvec_hbm, num_out, den_out,
                  s_v, src_v, dst_v, ae_v, cv_v, num_v, den_v):
        cid = lax.axis_index("c")
        sid = lax.axis_index("s")
        wid = sid * NC + cid
        base = wid * per
        pltpu.sync_copy(s_hbm, s_v)
        pltpu.sync_copy(ei_hbm.at[pl.ds(base, per)], src_v.at[pl.ds(0, per)])
        pltpu.sync_copy(ei_hbm.at[pl.ds(e + base, per)], dst_v.at[pl.ds(0, per)])
        pltpu.sync_copy(ae_hbm.at[pl.ds(base, per)], ae_v.at[pl.ds(0, per)])
        pltpu.sync_copy(cvec_hbm, cv_v)
        c16 = cv_v[...]

        def zbody(i, carry):
            num_v[pl.ds(i * 16, 16)] = jnp.zeros((16,), jnp.float32)
            den_v[pl.ds(i * 16, 16)] = jnp.zeros((16,), jnp.float32)
            return carry

        lax.fori_loop(0, n // 16, zbody, 0)

        def chunk(off, mask):
            s16 = src_v[pl.ds(off, 16)] * 8
            d16 = dst_v[pl.ds(off, 16)]
            ae16 = ae_v[pl.ds(off, 16)]
            asr = plsc.load_gather(s_v, [s16], mask=mask)
            ads = plsc.load_gather(s_v, [d16 * 8 + 1], mask=mask)
            z16 = plsc.load_gather(s_v, [s16 + 2], mask=mask)
            al = asr + ads + ae16
            al = jnp.where(al >= 0.0, al, al * 0.2)
            ex = jnp.exp(al - c16)
            plsc.addupdate_scatter(den_v, [d16], ex, mask=mask)
            plsc.addupdate_scatter(num_v, [d16], ex * z16, mask=mask)

        def ebody(i, carry):
            chunk(i * 16, None)
            return carry

        lax.fori_loop(0, nfull, ebody, 0)
        if tail:
            chunk(nfull * 16, lax.iota(jnp.int32, 16) < tail)
        pltpu.sync_copy(num_v, num_out.at[wid])
        pltpu.sync_copy(den_v, den_out.at[wid])

    return sc_kernel


def kernel(x, edge_index, edge_attr, W, att_src, att_dst, W_edge, att_edge,
           bias_conv, W_lin, b_lin):
    n, d_in = x.shape
    e = edge_index.shape[1]
    d_edge = edge_attr.shape[1]

    a_mat = jnp.concatenate(
        [att_src[:, None], att_dst[:, None], W_lin,
         jnp.zeros((d_in, 5), jnp.float32)], axis=1)
    blk_n = 1000
    s_out, aux1 = pl.pallas_call(
        _tc1_body,
        grid=(n // blk_n,),
        in_specs=[
            pl.BlockSpec((blk_n, d_in), lambda i: (i, 0)),
            pl.BlockSpec((d_in, d_in), lambda i: (0, 0)),
            pl.BlockSpec((d_in, 8), lambda i: (0, 0)),
        ],
        out_specs=[
            pl.BlockSpec((blk_n, 8), lambda i: (i, 0)),
            pl.BlockSpec((8, 128), lambda i: (0, 0)),
        ],
        out_shape=[
            jax.ShapeDtypeStruct((n, 8), jnp.float32),
            jax.ShapeDtypeStruct((8, 128), jnp.float32),
        ],
    )(x, W, a_mat)

    rows = e * d_edge // 128
    blk_r = rows // 20
    ae_out, aux2 = pl.pallas_call(
        _tc2_body,
        grid=(rows // blk_r,),
        in_specs=[
            pl.BlockSpec((blk_r, 128), lambda i: (i, 0)),
            pl.BlockSpec((d_edge, d_in), lambda i: (0, 0)),
            pl.BlockSpec((d_in, 1), lambda i: (0, 0)),
        ],
        out_specs=[
            pl.BlockSpec((blk_r, 8), lambda i: (i, 0)),
            pl.BlockSpec((8, 128), lambda i: (0, 0)),
        ],
        out_shape=[
            jax.ShapeDtypeStruct((rows, 8), jnp.float32),
            jax.ShapeDtypeStruct((8, 128), jnp.float32),
        ],
    )(edge_attr.reshape(rows, 128), W_edge, att_edge[:, None])

    c_shift = aux1[0, 0] + aux1[1, 0] + aux2[0, 0]
    cvec = jnp.broadcast_to(c_shift, (16,)).astype(jnp.float32)

    num_p, den_p = _make_sc_kernel(n, e)(
        s_out.reshape(8 * n), edge_index.reshape(2 * e), ae_out.reshape(e), cvec)

    out = pl.pallas_call(
        _tc3_body,
        out_shape=jax.ShapeDtypeStruct((1, n), jnp.float32),
    )(num_p, den_p, bias_conv.reshape(3, 128), W_lin.reshape(3, 128),
      b_lin.reshape(1, 1))
    return out.reshape(n, 1)


# speedup vs baseline: 42.8033x; 1.2883x over previous
import functools

import jax
import jax.numpy as jnp
from jax import lax
from jax.experimental import pallas as pl
from jax.experimental.pallas import tpu as pltpu
from jax.experimental.pallas import tpu_sc as plsc

NC = 2
NS = 16
NW = NC * NS


def _tc1_body(x_ref, w_ref, a_ref, asrc_ref, adst_ref, z_ref, aux_ref):
    v = jnp.dot(w_ref[...], a_ref[...], preferred_element_type=jnp.float32)
    st = lax.dot_general(v, x_ref[...], (((0,), (1,)), ((), ())),
                         preferred_element_type=jnp.float32)
    asrc_ref[...] = st[0, :]
    adst_ref[...] = st[1, :]
    z_ref[...] = st[2, :]
    m = jnp.max(st, axis=1).reshape(8, 1)
    aux_ref[...] = jnp.broadcast_to(m, (8, 128))


def _tc2_body(ea_ref, we_ref, ae_ref, o0, o1, o2, o3, o4, o5, o6, o7,
              aux_ref):
    ve = jnp.dot(we_ref[...], ae_ref[...], preferred_element_type=jnp.float32)
    vt = jnp.broadcast_to(ve.reshape(1, 16, 1), (8, 16, 1)).reshape(128, 1)
    row = lax.broadcasted_iota(jnp.int32, (128, 8), 0)
    col = lax.broadcasted_iota(jnp.int32, (128, 8), 1)
    vd = jnp.where(row // 16 == col, vt, 0.0)
    at8 = lax.dot_general(vd, ea_ref[...], (((0,), (1,)), ((), ())),
                          preferred_element_type=jnp.float32)
    o0[...] = at8[0, :]
    o1[...] = at8[1, :]
    o2[...] = at8[2, :]
    o3[...] = at8[3, :]
    o4[...] = at8[4, :]
    o5[...] = at8[5, :]
    o6[...] = at8[6, :]
    o7[...] = at8[7, :]
    aux_ref[...] = jnp.broadcast_to(jnp.max(at8), (8, 128))


def _tc3_body(num_ref, den_ref, bc_ref, wl_ref, bl_ref, out_ref):
    ns = jnp.sum(num_ref[...], axis=0, keepdims=True)
    ds_ = jnp.sum(den_ref[...], axis=0, keepdims=True)
    c = jnp.sum(bc_ref[...] * wl_ref[...]) + bl_ref[0, 0]
    agg = jnp.where(ds_ > 0.0, ns / ds_, 0.0)
    out_ref[...] = jnp.maximum(agg + c, 0.0)


def _make_sc_kernel(n, e):
    per = -(-e // NW)
    nfull = per // 16
    tail = per - nfull * 16
    buf = (per + 15) // 16 * 16
    rows8 = e // 8
    lrow = (per // 8 + 14) // 8 * 8
    mesh = plsc.VectorSubcoreMesh(core_axis_name="c", subcore_axis_name="s")

    @functools.partial(
        pl.kernel,
        mesh=mesh,
        compiler_params=pltpu.CompilerParams(needs_layout_passes=False),
        out_type=[
            jax.ShapeDtypeStruct((NW, n), jnp.float32),
            jax.ShapeDtypeStruct((NW, n), jnp.float32),
        ],
        scratch_types=[
            pltpu.VMEM((n,), jnp.float32),
            pltpu.VMEM((n,), jnp.float32),
            pltpu.VMEM((n,), jnp.float32),
            pltpu.VMEM((buf,), jnp.int32),
            pltpu.VMEM((buf,), jnp.int32),
            pltpu.VMEM((8 * lrow,), jnp.float32),
            pltpu.VMEM((16,), jnp.float32),
            pltpu.VMEM((n,), jnp.float32),
            pltpu.VMEM((n,), jnp.float32),
        ],
    )
    def sc_kernel(asrc_hbm, adst_hbm, z_hbm, ei_hbm, ae0, ae1, ae2, ae3,
                  ae4, ae5, ae6, ae7, cvec_hbm, num_out, den_out,
                  asrc_v, adst_v, z_v, src_v, dst_v, aeb_v, cv_v,
                  num_v, den_v):
        cid = lax.axis_index("c")
        sid = lax.axis_index("s")
        wid = sid * NC + cid
        base = wid * per
        pltpu.sync_copy(asrc_hbm, asrc_v)
        pltpu.sync_copy(adst_hbm, adst_v)
        pltpu.sync_copy(z_hbm, z_v)
        pltpu.sync_copy(ei_hbm.at[pl.ds(base, per)], src_v.at[pl.ds(0, per)])
        pltpu.sync_copy(ei_hbm.at[pl.ds(e + base, per)], dst_v.at[pl.ds(0, per)])
        q0 = base // 8
        d0 = q0 - q0 // 8 * 8
        s0 = jnp.minimum(q0 - d0, rows8 - lrow)
        for j, aej in enumerate((ae0, ae1, ae2, ae3, ae4, ae5, ae6, ae7)):
            pltpu.sync_copy(aej.at[pl.ds(s0, lrow)],
                            aeb_v.at[pl.ds(j * lrow, lrow)])
        pltpu.sync_copy(cvec_hbm, cv_v)
        c16 = cv_v[...]
        lane = lax.iota(jnp.int32, 16)
        aeidx = (lane & 7) * lrow + (lane >> 3) + (q0 - s0)

        zero = jnp.zeros((16,), jnp.float32)

        def zbody(i, carry):
            for j in range(4):
                num_v[pl.ds(i * 64 + j * 16, 16)] = zero
                den_v[pl.ds(i * 64 + j * 16, 16)] = zero
            return carry

        lax.fori_loop(0, n // 64, zbody, 0)
        for k in range(n // 64 * 64, n, 16):
            num_v[pl.ds(k, 16)] = zero
            den_v[pl.ds(k, 16)] = zero

        def chunk(off, mask):
            s16 = src_v[pl.ds(off, 16)]
            d16 = dst_v[pl.ds(off, 16)]
            ae16 = plsc.load_gather(aeb_v, [aeidx + off // 8], mask=mask)
            asr = plsc.load_gather(asrc_v, [s16], mask=mask)
            ads = plsc.load_gather(adst_v, [d16], mask=mask)
            z16 = plsc.load_gather(z_v, [s16], mask=mask)
            al = asr + ads + ae16
            al = jnp.where(al >= 0.0, al, al * 0.2)
            ex = jnp.exp(al - c16)
            plsc.addupdate_scatter(den_v, [d16], ex, mask=mask)
            plsc.addupdate_scatter(num_v, [d16], ex * z16, mask=mask)

        def ebody(i, carry):
            chunk(i * 32, None)
            chunk(i * 32 + 16, None)
            return carry

        lax.fori_loop(0, nfull // 2, ebody, 0)
        for k in range(nfull // 2 * 2, nfull):
            chunk(k * 16, None)
        if tail:
            chunk(nfull * 16, lane < tail)
        pltpu.sync_copy(num_v, num_out.at[wid])
        pltpu.sync_copy(den_v, den_out.at[wid])

    return sc_kernel


def kernel(x, edge_index, edge_attr, W, att_src, att_dst, W_edge, att_edge,
           bias_conv, W_lin, b_lin):
    n, d_in = x.shape
    e = edge_index.shape[1]
    d_edge = edge_attr.shape[1]

    a_mat = jnp.concatenate(
        [att_src[:, None], att_dst[:, None], W_lin,
         jnp.zeros((d_in, 5), jnp.float32)], axis=1)
    asrc, adst, zvec, aux1 = pl.pallas_call(
        _tc1_body,
        out_shape=[
            jax.ShapeDtypeStruct((n,), jnp.float32),
            jax.ShapeDtypeStruct((n,), jnp.float32),
            jax.ShapeDtypeStruct((n,), jnp.float32),
            jax.ShapeDtypeStruct((8, 128), jnp.float32),
        ],
    )(x, W, a_mat)

    rows8 = e // 8
    ae_outs = pl.pallas_call(
        _tc2_body,
        out_shape=[jax.ShapeDtypeStruct((rows8,), jnp.float32)] * 8
        + [jax.ShapeDtypeStruct((8, 128), jnp.float32)],
    )(edge_attr.reshape(rows8 * 8 * d_edge // 128, 128), W_edge,
      att_edge[:, None])
    aux2 = ae_outs[8]

    c_shift = aux1[0, 0] + aux1[1, 0] + aux2[0, 0]
    cvec = jnp.broadcast_to(c_shift, (16,)).astype(jnp.float32)

    num_p, den_p = _make_sc_kernel(n, e)(
        asrc, adst, zvec, edge_index.reshape(2 * e), *ae_outs[:8], cvec)

    out = pl.pallas_call(
        _tc3_body,
        out_shape=jax.ShapeDtypeStruct((1, n), jnp.float32),
    )(num_p, den_p, bias_conv.reshape(3, 128), W_lin.reshape(3, 128),
      b_lin.reshape(1, 1))
    return out.reshape(n, 1)


# speedup vs baseline: 45.2529x; 1.0572x over previous
import functools

import jax
import jax.numpy as jnp
from jax import lax
from jax.experimental import pallas as pl
from jax.experimental.pallas import tpu as pltpu
from jax.experimental.pallas import tpu_sc as plsc

NC = 2
NS = 16
NW = NC * NS


def _tc1_body(x_ref, w_ref, a_ref, asrc_ref, adst_ref, z_ref, aux_ref):
    v = jnp.dot(w_ref[...], a_ref[...], preferred_element_type=jnp.float32)
    st = lax.dot_general(v, x_ref[...], (((0,), (1,)), ((), ())),
                         preferred_element_type=jnp.float32)
    asrc_ref[...] = st[0, :]
    adst_ref[...] = st[1, :]
    z_ref[...] = st[2, :]
    m = jnp.max(st, axis=1).reshape(8, 1)
    aux_ref[...] = jnp.broadcast_to(m, (8, 128))


def _tc2_body(ea_ref, we_ref, ae_ref, o0, o1, o2, o3, o4, o5, o6, o7,
              aux_ref):
    ve = jnp.dot(we_ref[...], ae_ref[...], preferred_element_type=jnp.float32)
    vt = jnp.broadcast_to(ve.reshape(1, 16, 1), (8, 16, 1)).reshape(128, 1)
    row = lax.broadcasted_iota(jnp.int32, (128, 8), 0)
    col = lax.broadcasted_iota(jnp.int32, (128, 8), 1)
    vd = jnp.where(row // 16 == col, vt, 0.0)
    at8 = lax.dot_general(vd, ea_ref[...], (((0,), (1,)), ((), ())),
                          preferred_element_type=jnp.float32)
    o0[...] = at8[0, :]
    o1[...] = at8[1, :]
    o2[...] = at8[2, :]
    o3[...] = at8[3, :]
    o4[...] = at8[4, :]
    o5[...] = at8[5, :]
    o6[...] = at8[6, :]
    o7[...] = at8[7, :]
    aux_ref[...] = jnp.broadcast_to(jnp.max(at8), (8, 128))


def _tc3_body(num_ref, den_ref, bc_ref, wl_ref, bl_ref, out_ref):
    ns = jnp.sum(num_ref[...], axis=0, keepdims=True)
    ds_ = jnp.sum(den_ref[...], axis=0, keepdims=True)
    c = jnp.sum(bc_ref[...] * wl_ref[...]) + bl_ref[0, 0]
    agg = jnp.where(ds_ > 0.0, ns / ds_, 0.0)
    out_ref[...] = jnp.maximum(agg + c, 0.0)


def _make_sc_kernel(n, e):
    per = -(-e // NW)
    nfull = per // 16
    tail = per - nfull * 16
    buf = (per + 15) // 16 * 16
    rows8 = e // 8
    lrow = (per // 8 + 14) // 8 * 8
    mesh = plsc.VectorSubcoreMesh(core_axis_name="c", subcore_axis_name="s")

    @functools.partial(
        pl.kernel,
        mesh=mesh,
        compiler_params=pltpu.CompilerParams(needs_layout_passes=False),
        out_type=[
            jax.ShapeDtypeStruct((NW, n), jnp.float32),
            jax.ShapeDtypeStruct((NW, n), jnp.float32),
        ],
        scratch_types=[
            pltpu.VMEM((n,), jnp.float32),
            pltpu.VMEM((n,), jnp.float32),
            pltpu.VMEM((n,), jnp.float32),
            pltpu.VMEM((buf,), jnp.int32),
            pltpu.VMEM((buf,), jnp.int32),
            pltpu.VMEM((8 * lrow,), jnp.float32),
            pltpu.VMEM((16,), jnp.float32),
            pltpu.VMEM((n,), jnp.float32),
            pltpu.VMEM((n,), jnp.float32),
            pltpu.SemaphoreType.DMA,
        ],
    )
    def sc_kernel(asrc_hbm, adst_hbm, z_hbm, ei_hbm, ae0, ae1, ae2, ae3,
                  ae4, ae5, ae6, ae7, cvec_hbm, num_out, den_out,
                  asrc_v, adst_v, z_v, src_v, dst_v, aeb_v, cv_v,
                  num_v, den_v, dsem):
        cid = lax.axis_index("c")
        sid = lax.axis_index("s")
        wid = sid * NC + cid
        base = wid * per
        q0 = base // 8
        d0 = q0 - q0 // 8 * 8
        s0 = jnp.minimum(q0 - d0, rows8 - lrow)
        cps = [
            pltpu.async_copy(asrc_hbm, asrc_v, dsem),
            pltpu.async_copy(adst_hbm, adst_v, dsem),
            pltpu.async_copy(z_hbm, z_v, dsem),
            pltpu.async_copy(ei_hbm.at[pl.ds(base, per)],
                             src_v.at[pl.ds(0, per)], dsem),
            pltpu.async_copy(ei_hbm.at[pl.ds(e + base, per)],
                             dst_v.at[pl.ds(0, per)], dsem),
            pltpu.async_copy(cvec_hbm, cv_v, dsem),
        ]
        for j, aej in enumerate((ae0, ae1, ae2, ae3, ae4, ae5, ae6, ae7)):
            cps.append(pltpu.async_copy(aej.at[pl.ds(s0, lrow)],
                                        aeb_v.at[pl.ds(j * lrow, lrow)], dsem))
        lane = lax.iota(jnp.int32, 16)
        aeidx = (lane & 7) * lrow + (lane >> 3) + (q0 - s0)

        zero = jnp.zeros((16,), jnp.float32)

        def zbody(i, carry):
            for j in range(4):
                num_v[pl.ds(i * 64 + j * 16, 16)] = zero
                den_v[pl.ds(i * 64 + j * 16, 16)] = zero
            return carry

        lax.fori_loop(0, n // 64, zbody, 0)
        for k in range(n // 64 * 64, n, 16):
            num_v[pl.ds(k, 16)] = zero
            den_v[pl.ds(k, 16)] = zero
        for cp in cps:
            cp.wait()
        c16 = cv_v[...]

        def chunk(off, mask):
            s16 = src_v[pl.ds(off, 16)]
            d16 = dst_v[pl.ds(off, 16)]
            ae16 = plsc.load_gather(aeb_v, [aeidx + off // 8], mask=mask)
            asr = plsc.load_gather(asrc_v, [s16], mask=mask)
            ads = plsc.load_gather(adst_v, [d16], mask=mask)
            z16 = plsc.load_gather(z_v, [s16], mask=mask)
            al = asr + ads + ae16
            al = jnp.where(al >= 0.0, al, al * 0.2)
            ex = jnp.exp(al - c16)
            plsc.addupdate_scatter(den_v, [d16], ex, mask=mask)
            plsc.addupdate_scatter(num_v, [d16], ex * z16, mask=mask)

        def ebody(i, carry):
            chunk(i * 32, None)
            chunk(i * 32 + 16, None)
            return carry

        lax.fori_loop(0, nfull // 2, ebody, 0)
        for k in range(nfull // 2 * 2, nfull):
            chunk(k * 16, None)
        if tail:
            chunk(nfull * 16, lane < tail)
        pltpu.sync_copy(num_v, num_out.at[wid])
        pltpu.sync_copy(den_v, den_out.at[wid])

    return sc_kernel


def kernel(x, edge_index, edge_attr, W, att_src, att_dst, W_edge, att_edge,
           bias_conv, W_lin, b_lin):
    n, d_in = x.shape
    e = edge_index.shape[1]
    d_edge = edge_attr.shape[1]

    a_mat = jnp.concatenate(
        [att_src[:, None], att_dst[:, None], W_lin,
         jnp.zeros((d_in, 5), jnp.float32)], axis=1)
    asrc, adst, zvec, aux1 = pl.pallas_call(
        _tc1_body,
        out_shape=[
            jax.ShapeDtypeStruct((n,), jnp.float32),
            jax.ShapeDtypeStruct((n,), jnp.float32),
            jax.ShapeDtypeStruct((n,), jnp.float32),
            jax.ShapeDtypeStruct((8, 128), jnp.float32),
        ],
    )(x, W, a_mat)

    rows8 = e // 8
    ae_outs = pl.pallas_call(
        _tc2_body,
        out_shape=[jax.ShapeDtypeStruct((rows8,), jnp.float32)] * 8
        + [jax.ShapeDtypeStruct((8, 128), jnp.float32)],
    )(edge_attr.reshape(rows8 * 8 * d_edge // 128, 128), W_edge,
      att_edge[:, None])
    aux2 = ae_outs[8]

    c_shift = aux1[0, 0] + aux1[1, 0] + aux2[0, 0]
    cvec = jnp.broadcast_to(c_shift, (16,)).astype(jnp.float32)

    num_p, den_p = _make_sc_kernel(n, e)(
        asrc, adst, zvec, edge_index.reshape(2 * e), *ae_outs[:8], cvec)

    out = pl.pallas_call(
        _tc3_body,
        out_shape=jax.ShapeDtypeStruct((1, n), jnp.float32),
    )(num_p, den_p, bias_conv.reshape(3, 128), W_lin.reshape(3, 128),
      b_lin.reshape(1, 1))
    return out.reshape(n, 1)
